# baseline (speedup 1.0000x reference)
import functools

import jax
import jax.numpy as jnp
from jax import lax
from jax.experimental import pallas as pl
from jax.experimental.pallas import tpu as pltpu
from jax.experimental.pallas import tpu_sc as plsc

B = 4096
SEQ = 200
DIM = 64
ROWS = SEQ * DIM
L = 16
NC = 2
NS = 16
NW = NC * NS
BANDS = ROWS // 8
PW_BANDS = BANDS // NW
HALF = 2048
NCH = PW_BANDS * 2
NBUF = 4
PD = 2
GROUPS = NCH // NBUF

_mesh = plsc.VectorSubcoreMesh(core_axis_name="c", subcore_axis_name="s")


@functools.partial(
    pl.kernel,
    out_type=jax.ShapeDtypeStruct((ROWS, B), jnp.float32),
    mesh=_mesh,
    scratch_types=dict(
        pos_v=pltpu.VMEM((PW_BANDS * 8 * L,), jnp.float32),
        bufs=[pltpu.VMEM((8, HALF), jnp.float32) for _ in range(NBUF)],
        isems=[pltpu.SemaphoreType.DMA for _ in range(NBUF)],
        osems=[pltpu.SemaphoreType.DMA for _ in range(NBUF)],
    ),
)
def _sc_add(x_hbm, pos_hbm, out_hbm, *, pos_v, bufs, isems, osems):
    wid = lax.axis_index("s") * NC + lax.axis_index("c")

    pltpu.sync_copy(pos_hbm.at[pl.ds(wid * (PW_BANDS * 8 * L), PW_BANDS * 8 * L)],
                    pos_v)

    def chunk_slice(c):
        half = wid * NCH + c
        t = half >> 1
        h = half & 1
        r0 = pl.multiple_of(t * 8, 8)
        l0 = pl.multiple_of(h * HALF, HALF)
        return r0, l0

    def compute_chunk(buf, c):
        pb = (c >> 1) * (8 * L)
        pv = [pos_v[pl.ds(pb + dr * L, L)] for dr in range(8)]

        def body(bq, carry):
            base = pl.multiple_of(bq * 128, 128)
            for dr in range(8):
                for cc in range(8):
                    plsc.addupdate(buf.at[dr, pl.ds(base + cc * L, L)], pv[dr])
            return carry
        lax.fori_loop(0, HALF // 128, body, 0)

    def slot(c, k, *, osem_wait=True, prefetch=True):
        r0, l0 = chunk_slice(c)
        pltpu.make_async_copy(x_hbm.at[pl.ds(r0, 8), pl.ds(l0, HALF)],
                              bufs[k], isems[k]).wait()
        compute_chunk(bufs[k], c)
        pltpu.async_copy(bufs[k], out_hbm.at[pl.ds(r0, 8), pl.ds(l0, HALF)],
                         osems[k])
        if prefetch:
            kp = (k + PD) % NBUF
            if osem_wait:
                rp, lp = chunk_slice(c + PD - NBUF)
                pltpu.make_async_copy(
                    bufs[kp], out_hbm.at[pl.ds(rp, 8), pl.ds(lp, HALF)],
                    osems[kp]).wait()
            rn, ln = chunk_slice(c + PD)
            pltpu.async_copy(x_hbm.at[pl.ds(rn, 8), pl.ds(ln, HALF)],
                             bufs[kp], isems[kp])

    for k in range(PD):
        r0, l0 = chunk_slice(k)
        pltpu.async_copy(x_hbm.at[pl.ds(r0, 8), pl.ds(l0, HALF)],
                         bufs[k], isems[k])

    for k in range(NBUF):
        slot(k, k, osem_wait=(k + PD - NBUF >= 0))

    def group_body(g, carry):
        for k in range(NBUF):
            slot(g * NBUF + k, k)
        return carry
    lax.fori_loop(1, GROUPS - 1, group_body, 0)

    for k in range(NBUF):
        slot((GROUPS - 1) * NBUF + k, k, prefetch=(k < NBUF - PD))

    for j in range(NBUF):
        c = NCH - NBUF + j
        r0, l0 = chunk_slice(c)
        pltpu.make_async_copy(bufs[c % NBUF],
                              out_hbm.at[pl.ds(r0, 8), pl.ds(l0, HALF)],
                              osems[c % NBUF]).wait()


def kernel(x, pos_table):
    xt = x.transpose(1, 2, 0).reshape(ROWS, B)
    pos_rep = jnp.repeat(pos_table.reshape(-1), L, total_repeat_length=ROWS * L)
    out2 = _sc_add(xt, pos_rep)
    return out2.reshape(SEQ, DIM, B).transpose(2, 0, 1)
